# speedup vs baseline: 36.3130x; 36.3130x over previous
import jax
import jax.numpy as jnp
from jax.experimental import pallas as pl
from jax.experimental.pallas import tpu as pltpu


def _tpr_body(o0_ref, o1_ref, t_ref, out_ref):
    pred1 = o1_ref[...] > o0_ref[...]
    tpos = t_ref[...] == 1
    tp = jnp.sum(jnp.where(pred1 & tpos, 1.0, 0.0))
    pos = jnp.sum(jnp.where(tpos, 1.0, 0.0))
    out_ref[0, 0] = tp / (pos + 1e-10)


def kernel(output, target):
    B = output.shape[0]
    o0 = output[:, 0].reshape(B // 128, 128)
    o1 = output[:, 1].reshape(B // 128, 128)
    t = target.astype(jnp.int32).reshape(B // 128, 128)
    res = pl.pallas_call(
        _tpr_body,
        out_shape=jax.ShapeDtypeStruct((1, 1), jnp.float32),
        out_specs=pl.BlockSpec(memory_space=pltpu.SMEM),
    )(o0, o1, t)
    return res[0, 0]
